# Initial kernel scaffold; baseline (speedup 1.0000x reference)
import jax
import jax.numpy as jnp
from jax.experimental import pallas as pl


def kernel(h, edge_index, W1, b1, W2, b2, Wm, bm):
    raise NotImplementedError("write your pallas kernel here")


# speedup vs baseline: 8.9979x; 8.9979x over previous
import functools

import jax
import jax.numpy as jnp
from jax import lax
from jax.experimental import pallas as pl
from jax.experimental.pallas import tpu as pltpu
from jax.experimental.pallas import tpu_sc as plsc

NC = 2
NS = 16
NW = NC * NS
CHUNK = 128


def _mesh():
  return plsc.VectorSubcoreMesh(
      core_axis_name="c", subcore_axis_name="s",
      num_cores=NC, num_subcores=NS)


def _slice_split(n):
  for k in range(NS, 0, -1):
    if n % k == 0 and (n // k) % 8 == 0:
      return k, n // k
  raise ValueError(n)


def _make_scatter(n, f, nchunks, with_deg, interpret=False):
  kz, rpt = _slice_split(n)

  out_type = [jax.ShapeDtypeStruct((NC, n, f), jnp.float32)]
  scratch = [
      pltpu.VMEM((CHUNK,), jnp.int32),
      pltpu.VMEM((CHUNK,), jnp.int32),
      pltpu.VMEM((CHUNK, f), jnp.float32),
      pltpu.VMEM_SHARED((n, f), jnp.float32),
      pltpu.SemaphoreType.DMA,
  ]
  if with_deg:
    out_type.append(jax.ShapeDtypeStruct((NC, n, f), jnp.float32))
    scratch += [
        pltpu.VMEM((CHUNK, f), jnp.float32),
        pltpu.VMEM_SHARED((n, f), jnp.float32),
    ]

  def body(*refs):
    if with_deg:
      (feat, srcc, dstc, zf, ones_h,
       agg_out, deg_out,
       src_v, dst_v, rows_v, acc_sh, sem, ones_v, deg_sh) = refs
    else:
      (feat, srcc, dstc, zf,
       agg_out,
       src_v, dst_v, rows_v, acc_sh, sem) = refs

    cid = lax.axis_index("c")
    sid = lax.axis_index("s")
    wid = cid * NS + sid
    r0 = sid * rpt

    @pl.when(sid < kz)
    def _zero():
      pltpu.sync_copy(zf, acc_sh.at[pl.ds(r0, rpt), :])
      if with_deg:
        pltpu.sync_copy(zf, deg_sh.at[pl.ds(r0, rpt), :])

    if with_deg:
      pltpu.sync_copy(ones_h, ones_v)
    plsc.subcore_barrier()

    lo = (wid * nchunks) // NW
    hi = ((wid + 1) * nchunks) // NW

    def step(j, carry):
      pltpu.sync_copy(srcc.at[j], src_v)
      pltpu.sync_copy(dstc.at[j], dst_v)
      pltpu.async_copy(feat.at[src_v], rows_v, sem).wait()
      pltpu.sync_copy(rows_v, acc_sh.at[dst_v], add=True)
      if with_deg:
        pltpu.sync_copy(ones_v, deg_sh.at[dst_v], add=True)
      return carry

    lax.fori_loop(lo, hi, step, 0)
    plsc.subcore_barrier()

    @pl.when(sid < kz)
    def _writeout():
      pltpu.sync_copy(acc_sh.at[pl.ds(r0, rpt), :],
                      agg_out.at[cid, pl.ds(r0, rpt), :])
      if with_deg:
        pltpu.sync_copy(deg_sh.at[pl.ds(r0, rpt), :],
                        deg_out.at[cid, pl.ds(r0, rpt), :])

  return pl.kernel(body, out_type=tuple(out_type), mesh=_mesh(),
                   scratch_types=tuple(scratch), interpret=interpret,
                   compiler_params=pltpu.CompilerParams(
                       use_tc_tiling_on_sc=False,
                       needs_layout_passes=False))


def _make_score(n, nchunks, interpret=False):
  out_type = jax.ShapeDtypeStruct((nchunks, CHUNK), jnp.float32)
  scratch = (
      pltpu.VMEM((n,), jnp.float32),
      pltpu.VMEM((n,), jnp.float32),
      pltpu.VMEM((CHUNK,), jnp.int32),
      pltpu.VMEM((CHUNK,), jnp.int32),
      pltpu.VMEM((CHUNK,), jnp.float32),
  )

  def body(s0_h, s1_h, srcc, dstc, out_h, s0_v, s1_v, src_v, dst_v, out_v):
    cid = lax.axis_index("c")
    sid = lax.axis_index("s")
    wid = cid * NS + sid
    pltpu.sync_copy(s0_h, s0_v)
    pltpu.sync_copy(s1_h, s1_v)

    lo = (wid * nchunks) // NW
    hi = ((wid + 1) * nchunks) // NW

    def step(j, carry):
      pltpu.sync_copy(srcc.at[j], src_v)
      pltpu.sync_copy(dstc.at[j], dst_v)
      for k in range(CHUNK // 16):
        sv = src_v[pl.ds(k * 16, 16)]
        dv = dst_v[pl.ds(k * 16, 16)]
        a = plsc.load_gather(s0_v, [sv])
        b = plsc.load_gather(s1_v, [dv])
        out_v[pl.ds(k * 16, 16)] = a + b
      pltpu.sync_copy(out_v, out_h.at[j])
      return carry

    lax.fori_loop(lo, hi, step, 0)

  return pl.kernel(body, out_type=out_type, mesh=_mesh(),
                   scratch_types=scratch, interpret=interpret,
                   compiler_params=pltpu.CompilerParams(
                       use_tc_tiling_on_sc=False,
                       needs_layout_passes=False))


def _tc_project(h, w1):
  n, d = h.shape
  f = w1.shape[1]
  bn = 2000

  def body(h_ref, w_ref, o_ref):
    o_ref[...] = jnp.dot(h_ref[...], w_ref[...],
                         preferred_element_type=jnp.float32,
                         precision=jax.lax.Precision.HIGHEST)

  return pl.pallas_call(
      body,
      grid=(n // bn,),
      in_specs=[pl.BlockSpec((bn, d), lambda i: (i, 0)),
                pl.BlockSpec((d, f), lambda i: (0, 0))],
      out_specs=pl.BlockSpec((bn, f), lambda i: (i, 0)),
      out_shape=jax.ShapeDtypeStruct((n, f), jnp.float32),
  )(h, w1)


def _tc_layer1(aggp, degp, p1, b1):
  n, f = p1.shape

  def body(aggp_ref, degp_ref, p1_ref, b1_ref, hh_ref, inv_ref):
    deg = degp_ref[0, :, 0:1] + degp_ref[1, :, 0:1]
    inv = 1.0 / (deg + 1.0)
    hn = (aggp_ref[0] + aggp_ref[1] + p1_ref[...]) * inv
    hh_ref[...] = jnp.maximum(hn + b1_ref[...], 0.0)
    inv_ref[...] = inv

  return pl.pallas_call(
      body,
      out_shape=(jax.ShapeDtypeStruct((n, f), jnp.float32),
                 jax.ShapeDtypeStruct((n, 1), jnp.float32)),
  )(aggp, degp, p1, b1)


def _tc_layer2(agg2p, hh, inv, w2, wm, b2, bm):
  n, f = hh.shape
  d = w2.shape[1]

  def body(agg2p_ref, hh_ref, inv_ref, w2_ref, wm_ref, b2_ref, bm_ref,
           s0_ref, s1_ref):
    hn2 = (agg2p_ref[0] + agg2p_ref[1] + hh_ref[...]) * inv_ref[...]
    wm_top = wm_ref[:d, :]
    wm_bot = wm_ref[d:, :]
    u0 = jnp.dot(w2_ref[...], wm_top, preferred_element_type=jnp.float32,
                  precision=jax.lax.Precision.HIGHEST)
    u1 = jnp.dot(w2_ref[...], wm_bot, preferred_element_type=jnp.float32,
                  precision=jax.lax.Precision.HIGHEST)
    c0 = jnp.dot(b2_ref[...], wm_top, preferred_element_type=jnp.float32,
                  precision=jax.lax.Precision.HIGHEST)
    c1 = jnp.dot(b2_ref[...], wm_bot, preferred_element_type=jnp.float32,
                 precision=jax.lax.Precision.HIGHEST)
    s0_ref[...] = jnp.dot(hn2, u0, preferred_element_type=jnp.float32,
                          precision=jax.lax.Precision.HIGHEST) \
        + c0[0, 0] + bm_ref[0, 0]
    s1_ref[...] = jnp.dot(hn2, u1, preferred_element_type=jnp.float32,
                          precision=jax.lax.Precision.HIGHEST) \
        + c1[0, 0]

  return pl.pallas_call(
      body,
      out_shape=(jax.ShapeDtypeStruct((n, 1), jnp.float32),
                 jax.ShapeDtypeStruct((n, 1), jnp.float32)),
  )(agg2p, hh, inv, w2, wm, b2, bm)


@jax.jit
def kernel(h, edge_index, W1, b1, W2, b2, Wm, bm):
  n, d = h.shape
  e = edge_index.shape[1]
  f = W1.shape[1]
  nchunks = e // CHUNK
  _, rpt = _slice_split(n)

  srcc = edge_index[0].reshape(nchunks, CHUNK)
  dstc = edge_index[1].reshape(nchunks, CHUNK)
  zf = jnp.zeros((rpt, f), jnp.float32)
  ones = jnp.ones((CHUNK, f), jnp.float32)

  p1 = _tc_project(h, W1)
  aggp, degp = _make_scatter(n, f, nchunks, True)(p1, srcc, dstc, zf, ones)
  hh, inv = _tc_layer1(aggp, degp, p1, b1.reshape(1, f))
  (agg2p,) = _make_scatter(n, f, nchunks, False)(hh, srcc, dstc, zf)
  s0, s1 = _tc_layer2(agg2p, hh, inv, W2, Wm,
                      b2.reshape(1, d), bm.reshape(1, 1))
  score = _make_score(n, nchunks)(s0.reshape(n), s1.reshape(n), srcc, dstc)
  return score.reshape(e, 1)


# speedup vs baseline: 15.1927x; 1.6885x over previous
import jax
import jax.numpy as jnp
from jax import lax
from jax.experimental import pallas as pl
from jax.experimental.pallas import tpu as pltpu
from jax.experimental.pallas import tpu_sc as plsc

NC = 2
NS = 16
NW = NC * NS
CHUNK = 128
SB = 8
NPAD = 8


def _mesh():
  return plsc.VectorSubcoreMesh(
      core_axis_name="c", subcore_axis_name="s",
      num_cores=NC, num_subcores=NS)


def _sc_params():
  return pltpu.CompilerParams(
      use_tc_tiling_on_sc=False, needs_layout_passes=False)


def _slice_split(n):
  for k in range(NS, 0, -1):
    if n % k == 0 and (n // k) % 8 == 0:
      return k, n // k
  raise ValueError(n)


def _make_scatter(np_, f, nchp, with_deg, interpret=False):
  kz, rpt = _slice_split(np_)
  supt = nchp // (NW * SB)

  out_type = [jax.ShapeDtypeStruct((NC, np_, f), jnp.float32)]
  scratch = [
      pltpu.VMEM((SB, 2, CHUNK), jnp.int32),
      pltpu.VMEM((SB, CHUNK, f), jnp.float32),
      pltpu.VMEM_SHARED((np_, f), jnp.float32),
      pltpu.SemaphoreType.DMA,
      pltpu.SemaphoreType.DMA,
  ]
  if with_deg:
    out_type.append(jax.ShapeDtypeStruct((NC, np_, f), jnp.float32))
    scratch += [
        pltpu.VMEM((CHUNK, f), jnp.float32),
        pltpu.VMEM_SHARED((np_, f), jnp.float32),
    ]

  def body(*refs):
    if with_deg:
      (feat, eidx, zf, ones_h,
       agg_out, deg_out,
       idx_v, rows_v, acc_sh, gsem, ssem, ones_v, deg_sh) = refs
    else:
      (feat, eidx, zf,
       agg_out,
       idx_v, rows_v, acc_sh, gsem, ssem) = refs

    cid = lax.axis_index("c")
    sid = lax.axis_index("s")
    wid = cid * NS + sid
    r0 = sid * rpt

    @pl.when(sid < kz)
    def _zero():
      pltpu.sync_copy(zf, acc_sh.at[pl.ds(r0, rpt), :])
      if with_deg:
        pltpu.sync_copy(zf, deg_sh.at[pl.ds(r0, rpt), :])

    if with_deg:
      pltpu.sync_copy(ones_h, ones_v)
    plsc.subcore_barrier()

    def sup(s, carry):
      pltpu.sync_copy(eidx.at[pl.ds(s * SB, SB)], idx_v)
      gd = [pltpu.async_copy(feat.at[idx_v.at[k, 0]], rows_v.at[k], gsem)
            for k in range(SB)]
      sd = []
      for k in range(SB):
        gd[k].wait()
        sd.append(pltpu.async_copy(rows_v.at[k], acc_sh.at[idx_v.at[k, 1]],
                                   ssem, add=True))
        if with_deg:
          sd.append(pltpu.async_copy(ones_v, deg_sh.at[idx_v.at[k, 1]],
                                     ssem, add=True))
      for d in sd:
        d.wait()
      return carry

    lax.fori_loop(wid * supt, (wid + 1) * supt, sup, 0)
    plsc.subcore_barrier()

    @pl.when(sid < kz)
    def _writeout():
      pltpu.sync_copy(acc_sh.at[pl.ds(r0, rpt), :],
                      agg_out.at[cid, pl.ds(r0, rpt), :])
      if with_deg:
        pltpu.sync_copy(deg_sh.at[pl.ds(r0, rpt), :],
                        deg_out.at[cid, pl.ds(r0, rpt), :])

  return pl.kernel(body, out_type=tuple(out_type), mesh=_mesh(),
                   scratch_types=tuple(scratch), interpret=interpret,
                   compiler_params=_sc_params())


def _make_score(np_, nchp, interpret=False):
  supt = nchp // (NW * SB)
  out_type = jax.ShapeDtypeStruct((nchp, CHUNK), jnp.float32)
  scratch = (
      pltpu.VMEM((np_,), jnp.float32),
      pltpu.VMEM((np_,), jnp.float32),
      pltpu.VMEM((SB, 2, CHUNK), jnp.int32),
      pltpu.VMEM((SB, CHUNK), jnp.float32),
  )

  def body(s0_h, s1_h, eidx, out_h, s0_v, s1_v, idx_v, out_v):
    cid = lax.axis_index("c")
    sid = lax.axis_index("s")
    wid = cid * NS + sid
    pltpu.sync_copy(s0_h, s0_v)
    pltpu.sync_copy(s1_h, s1_v)

    def sup(s, carry):
      pltpu.sync_copy(eidx.at[pl.ds(s * SB, SB)], idx_v)
      for k in range(SB):
        for i in range(CHUNK // 16):
          sv = idx_v[k, 0, pl.ds(i * 16, 16)]
          dv = idx_v[k, 1, pl.ds(i * 16, 16)]
          a = plsc.load_gather(s0_v, [sv])
          b = plsc.load_gather(s1_v, [dv])
          out_v[k, pl.ds(i * 16, 16)] = a + b
      pltpu.sync_copy(out_v, out_h.at[pl.ds(s * SB, SB), :])
      return carry

    lax.fori_loop(wid * supt, (wid + 1) * supt, sup, 0)

  return pl.kernel(body, out_type=out_type, mesh=_mesh(),
                   scratch_types=scratch, interpret=interpret,
                   compiler_params=_sc_params())


def _tc_project(h, w1):
  n, d = h.shape
  f = w1.shape[1]
  bn = 2000

  def body(h_ref, w_ref, o_ref):
    o_ref[...] = jnp.dot(h_ref[...], w_ref[...],
                         preferred_element_type=jnp.float32,
                         precision=jax.lax.Precision.HIGHEST)

  return pl.pallas_call(
      body,
      grid=(n // bn,),
      in_specs=[pl.BlockSpec((bn, d), lambda i: (i, 0)),
                pl.BlockSpec((d, f), lambda i: (0, 0))],
      out_specs=pl.BlockSpec((bn, f), lambda i: (i, 0)),
      out_shape=jax.ShapeDtypeStruct((n, f), jnp.float32),
  )(h, w1)


def _tc_layer1(aggp, degp, p1p, b1):
  np_, f = p1p.shape

  def body(aggp_ref, degp_ref, p1_ref, b1_ref, hh_ref, inv_ref):
    deg = degp_ref[0, :, 0:1] + degp_ref[1, :, 0:1]
    inv = 1.0 / (deg + 1.0)
    hn = (aggp_ref[0] + aggp_ref[1] + p1_ref[...]) * inv
    hh_ref[...] = jnp.maximum(hn + b1_ref[...], 0.0)
    inv_ref[...] = inv

  return pl.pallas_call(
      body,
      out_shape=(jax.ShapeDtypeStruct((np_, f), jnp.float32),
                 jax.ShapeDtypeStruct((np_, 1), jnp.float32)),
  )(aggp, degp, p1p, b1)


def _tc_layer2(agg2p, hh, inv, w2, wm, b2, bm):
  np_, f = hh.shape
  d = w2.shape[1]

  def body(agg2p_ref, hh_ref, inv_ref, w2_ref, wm_ref, b2_ref, bm_ref,
           s0_ref, s1_ref):
    hn2 = (agg2p_ref[0] + agg2p_ref[1] + hh_ref[...]) * inv_ref[...]
    wm_top = wm_ref[:d, :]
    wm_bot = wm_ref[d:, :]
    hi = jax.lax.Precision.HIGHEST
    u0 = jnp.dot(w2_ref[...], wm_top, preferred_element_type=jnp.float32,
                 precision=hi)
    u1 = jnp.dot(w2_ref[...], wm_bot, preferred_element_type=jnp.float32,
                 precision=hi)
    c0 = jnp.dot(b2_ref[...], wm_top, preferred_element_type=jnp.float32,
                 precision=hi)
    c1 = jnp.dot(b2_ref[...], wm_bot, preferred_element_type=jnp.float32,
                 precision=hi)
    s0_ref[...] = jnp.dot(hn2, u0, preferred_element_type=jnp.float32,
                          precision=hi) + c0[0, 0] + bm_ref[0, 0]
    s1_ref[...] = jnp.dot(hn2, u1, preferred_element_type=jnp.float32,
                          precision=hi) + c1[0, 0]

  return pl.pallas_call(
      body,
      out_shape=(jax.ShapeDtypeStruct((np_, 1), jnp.float32),
                 jax.ShapeDtypeStruct((np_, 1), jnp.float32)),
  )(agg2p, hh, inv, w2, wm, b2, bm)


@jax.jit
def kernel(h, edge_index, W1, b1, W2, b2, Wm, bm):
  n, d = h.shape
  e = edge_index.shape[1]
  f = W1.shape[1]
  np_ = n + NPAD
  step = NW * SB * CHUNK
  ep = ((e + step - 1) // step) * step
  nchp = ep // CHUNK
  _, rpt = _slice_split(np_)

  pad = jnp.full((ep - e,), n, jnp.int32)
  srcp = jnp.concatenate([edge_index[0], pad]).reshape(nchp, CHUNK)
  dstp = jnp.concatenate([edge_index[1], pad]).reshape(nchp, CHUNK)
  eidx = jnp.stack([srcp, dstp], axis=1)
  zf = jnp.zeros((rpt, f), jnp.float32)
  ones = jnp.ones((CHUNK, f), jnp.float32)

  p1 = jnp.pad(_tc_project(h, W1), ((0, NPAD), (0, 0)))
  aggp, degp = _make_scatter(np_, f, nchp, True)(p1, eidx, zf, ones)
  hh, inv = _tc_layer1(aggp, degp, p1, b1.reshape(1, f))
  (agg2p,) = _make_scatter(np_, f, nchp, False)(hh, eidx, zf)
  s0, s1 = _tc_layer2(agg2p, hh, inv, W2, Wm,
                      b2.reshape(1, d), bm.reshape(1, 1))
  score = _make_score(np_, nchp)(s0.reshape(np_), s1.reshape(np_), eidx)
  return score.reshape(ep, 1)[:e]


# speedup vs baseline: 15.4238x; 1.0152x over previous
import jax
import jax.numpy as jnp
from jax import lax
from jax.experimental import pallas as pl
from jax.experimental.pallas import tpu as pltpu
from jax.experimental.pallas import tpu_sc as plsc

NC = 2
NS = 16
NW = NC * NS
CHUNK = 128
SB = 8
NPAD = 8


def _mesh():
  return plsc.VectorSubcoreMesh(
      core_axis_name="c", subcore_axis_name="s",
      num_cores=NC, num_subcores=NS)


def _sc_params():
  return pltpu.CompilerParams(
      use_tc_tiling_on_sc=False, needs_layout_passes=False)


def _slice_split(n):
  for k in range(NS, 0, -1):
    if n % k == 0 and (n // k) % 8 == 0:
      return k, n // k
  raise ValueError(n)


def _make_scatter(np_, f, nchp, with_deg, interpret=False):
  kz, rpt = _slice_split(np_)
  supt = nchp // (NW * SB)

  out_type = [jax.ShapeDtypeStruct((NC, np_, f), jnp.float32)]
  scratch = [
      pltpu.VMEM((2, SB, 2, CHUNK), jnp.int32),
      pltpu.VMEM((2, SB, CHUNK, f), jnp.float32),
      pltpu.VMEM_SHARED((np_, f), jnp.float32),
      pltpu.SemaphoreType.DMA,
      pltpu.SemaphoreType.DMA,
  ]
  if with_deg:
    out_type.append(jax.ShapeDtypeStruct((NC, np_, f), jnp.float32))
    scratch += [
        pltpu.VMEM((CHUNK, f), jnp.float32),
        pltpu.VMEM_SHARED((np_, f), jnp.float32),
    ]

  def body(*refs):
    if with_deg:
      (feat, eidx, zf, ones_h,
       agg_out, deg_out,
       idx_v, rows_v, acc_sh, gsem, ssem, ones_v, deg_sh) = refs
    else:
      (feat, eidx, zf,
       agg_out,
       idx_v, rows_v, acc_sh, gsem, ssem) = refs

    cid = lax.axis_index("c")
    sid = lax.axis_index("s")
    wid = cid * NS + sid
    r0 = sid * rpt

    @pl.when(sid < kz)
    def _zero():
      pltpu.sync_copy(zf, acc_sh.at[pl.ds(r0, rpt), :])
      if with_deg:
        pltpu.sync_copy(zf, deg_sh.at[pl.ds(r0, rpt), :])

    if with_deg:
      pltpu.sync_copy(ones_h, ones_v)
    plsc.subcore_barrier()

    def idx_copy(s, b):
      pltpu.sync_copy(eidx.at[pl.ds(s * SB, SB)], idx_v.at[b])

    def fire_gathers(b):
      for k in range(SB):
        pltpu.async_copy(feat.at[idx_v.at[b, k, 0]], rows_v.at[b, k], gsem)

    def wait_gathers():
      for k in range(SB):
        pltpu.make_async_copy(feat.at[pl.ds(0, CHUNK), :],
                              rows_v.at[0, k], gsem).wait()

    def fire_scatters(b):
      for k in range(SB):
        pltpu.async_copy(rows_v.at[b, k], acc_sh.at[idx_v.at[b, k, 1]],
                         ssem, add=True)
        if with_deg:
          pltpu.async_copy(ones_v, deg_sh.at[idx_v.at[b, k, 1]],
                           ssem, add=True)

    def wait_scatters():
      for _ in range(2 * SB if with_deg else SB):
        pltpu.make_async_copy(feat.at[pl.ds(0, CHUNK), :],
                              rows_v.at[0, 0], ssem).wait()

    s0 = wid * supt
    idx_copy(s0, 0)
    fire_gathers(0)
    wait_gathers()
    fire_scatters(0)
    idx_copy(s0 + 1, 1)
    fire_gathers(1)

    def pair(g, carry):
      s = s0 + 1 + 2 * g
      wait_gathers()
      fire_scatters(1)
      wait_scatters()
      idx_copy(s + 1, 0)
      fire_gathers(0)
      wait_gathers()
      fire_scatters(0)
      wait_scatters()
      idx_copy(s + 2, 1)
      fire_gathers(1)
      return carry

    lax.fori_loop(0, (supt - 2) // 2, pair, 0)
    wait_gathers()
    fire_scatters(1)
    wait_scatters()
    wait_scatters()
    plsc.subcore_barrier()

    @pl.when(sid < kz)
    def _writeout():
      pltpu.sync_copy(acc_sh.at[pl.ds(r0, rpt), :],
                      agg_out.at[cid, pl.ds(r0, rpt), :])
      if with_deg:
        pltpu.sync_copy(deg_sh.at[pl.ds(r0, rpt), :],
                        deg_out.at[cid, pl.ds(r0, rpt), :])

  return pl.kernel(body, out_type=tuple(out_type), mesh=_mesh(),
                   scratch_types=tuple(scratch), interpret=interpret,
                   compiler_params=_sc_params())


def _make_score(np_, nchp, interpret=False):
  supt = nchp // (NW * SB)
  out_type = jax.ShapeDtypeStruct((nchp, CHUNK), jnp.float32)
  scratch = (
      pltpu.VMEM((np_,), jnp.float32),
      pltpu.VMEM((np_,), jnp.float32),
      pltpu.VMEM((SB, 2, CHUNK), jnp.int32),
      pltpu.VMEM((SB, CHUNK), jnp.float32),
  )

  def body(s0_h, s1_h, eidx, out_h, s0_v, s1_v, idx_v, out_v):
    cid = lax.axis_index("c")
    sid = lax.axis_index("s")
    wid = cid * NS + sid
    pltpu.sync_copy(s0_h, s0_v)
    pltpu.sync_copy(s1_h, s1_v)

    def sup(s, carry):
      pltpu.sync_copy(eidx.at[pl.ds(s * SB, SB)], idx_v)
      for k in range(SB):
        for i in range(CHUNK // 16):
          sv = idx_v[k, 0, pl.ds(i * 16, 16)]
          dv = idx_v[k, 1, pl.ds(i * 16, 16)]
          a = plsc.load_gather(s0_v, [sv])
          b = plsc.load_gather(s1_v, [dv])
          out_v[k, pl.ds(i * 16, 16)] = a + b
      pltpu.sync_copy(out_v, out_h.at[pl.ds(s * SB, SB), :])
      return carry

    lax.fori_loop(wid * supt, (wid + 1) * supt, sup, 0)

  return pl.kernel(body, out_type=out_type, mesh=_mesh(),
                   scratch_types=scratch, interpret=interpret,
                   compiler_params=_sc_params())


def _tc_project(h, w1):
  n, d = h.shape
  f = w1.shape[1]
  bn = 2000

  def body(h_ref, w_ref, o_ref):
    o_ref[...] = jnp.dot(h_ref[...], w_ref[...],
                         preferred_element_type=jnp.float32,
                         precision=jax.lax.Precision.HIGHEST)

  return pl.pallas_call(
      body,
      grid=(n // bn,),
      in_specs=[pl.BlockSpec((bn, d), lambda i: (i, 0)),
                pl.BlockSpec((d, f), lambda i: (0, 0))],
      out_specs=pl.BlockSpec((bn, f), lambda i: (i, 0)),
      out_shape=jax.ShapeDtypeStruct((n, f), jnp.float32),
  )(h, w1)


def _tc_layer1(aggp, degp, p1p, b1):
  np_, f = p1p.shape

  def body(aggp_ref, degp_ref, p1_ref, b1_ref, hh_ref, inv_ref):
    deg = degp_ref[0, :, 0:1] + degp_ref[1, :, 0:1]
    inv = 1.0 / (deg + 1.0)
    hn = (aggp_ref[0] + aggp_ref[1] + p1_ref[...]) * inv
    hh_ref[...] = jnp.maximum(hn + b1_ref[...], 0.0)
    inv_ref[...] = inv

  return pl.pallas_call(
      body,
      out_shape=(jax.ShapeDtypeStruct((np_, f), jnp.float32),
                 jax.ShapeDtypeStruct((np_, 1), jnp.float32)),
  )(aggp, degp, p1p, b1)


def _tc_layer2(agg2p, hh, inv, w2, wm, b2, bm):
  np_, f = hh.shape
  d = w2.shape[1]

  def body(agg2p_ref, hh_ref, inv_ref, w2_ref, wm_ref, b2_ref, bm_ref,
           s0_ref, s1_ref):
    hn2 = (agg2p_ref[0] + agg2p_ref[1] + hh_ref[...]) * inv_ref[...]
    wm_top = wm_ref[:d, :]
    wm_bot = wm_ref[d:, :]
    hi = jax.lax.Precision.HIGHEST
    u0 = jnp.dot(w2_ref[...], wm_top, preferred_element_type=jnp.float32,
                 precision=hi)
    u1 = jnp.dot(w2_ref[...], wm_bot, preferred_element_type=jnp.float32,
                 precision=hi)
    c0 = jnp.dot(b2_ref[...], wm_top, preferred_element_type=jnp.float32,
                 precision=hi)
    c1 = jnp.dot(b2_ref[...], wm_bot, preferred_element_type=jnp.float32,
                 precision=hi)
    s0_ref[...] = jnp.dot(hn2, u0, preferred_element_type=jnp.float32,
                          precision=hi) + c0[0, 0] + bm_ref[0, 0]
    s1_ref[...] = jnp.dot(hn2, u1, preferred_element_type=jnp.float32,
                          precision=hi) + c1[0, 0]

  return pl.pallas_call(
      body,
      out_shape=(jax.ShapeDtypeStruct((np_, 1), jnp.float32),
                 jax.ShapeDtypeStruct((np_, 1), jnp.float32)),
  )(agg2p, hh, inv, w2, wm, b2, bm)


@jax.jit
def kernel(h, edge_index, W1, b1, W2, b2, Wm, bm):
  n, d = h.shape
  e = edge_index.shape[1]
  f = W1.shape[1]
  np_ = n + NPAD
  step = NW * SB * CHUNK
  ep = ((e + step - 1) // step) * step
  nchp = ep // CHUNK
  _, rpt = _slice_split(np_)

  pad = jnp.full((ep - e,), n, jnp.int32)
  srcp = jnp.concatenate([edge_index[0], pad]).reshape(nchp, CHUNK)
  dstp = jnp.concatenate([edge_index[1], pad]).reshape(nchp, CHUNK)
  eidx = jnp.stack([srcp, dstp], axis=1)
  zf = jnp.zeros((rpt, f), jnp.float32)
  ones = jnp.ones((CHUNK, f), jnp.float32)

  p1 = jnp.pad(_tc_project(h, W1), ((0, NPAD), (0, 0)))
  aggp, degp = _make_scatter(np_, f, nchp, True)(p1, eidx, zf, ones)
  hh, inv = _tc_layer1(aggp, degp, p1, b1.reshape(1, f))
  (agg2p,) = _make_scatter(np_, f, nchp, False)(hh, eidx, zf)
  s0, s1 = _tc_layer2(agg2p, hh, inv, W2, Wm,
                      b2.reshape(1, d), bm.reshape(1, 1))
  score = _make_score(np_, nchp)(s0.reshape(np_), s1.reshape(np_), eidx)
  return score.reshape(ep, 1)[:e]


# speedup vs baseline: 15.8674x; 1.0288x over previous
import jax
import jax.numpy as jnp
from jax import lax
from jax.experimental import pallas as pl
from jax.experimental.pallas import tpu as pltpu
from jax.experimental.pallas import tpu_sc as plsc

NC = 2
NS = 16
NW = NC * NS
CH = 2048
NPAD = 8


def _mesh():
  return plsc.VectorSubcoreMesh(
      core_axis_name="c", subcore_axis_name="s",
      num_cores=NC, num_subcores=NS)


def _sc_params():
  return pltpu.CompilerParams(
      use_tc_tiling_on_sc=False, needs_layout_passes=False)


def _slice_split(n):
  for k in range(NS, 0, -1):
    if n % k == 0 and (n // k) % 8 == 0:
      return k, n // k
  raise ValueError(n)


def _make_scatter(n, np_, f, ep, with_deg, interpret=False):
  kz, rpt = _slice_split(np_)
  cpt = ep // (NW * CH)

  out_type = [jax.ShapeDtypeStruct((NC, np_, f), jnp.float32)]
  scratch = [
      pltpu.VMEM((2, 2, CH), jnp.int32),
      pltpu.VMEM((2, CH, f), jnp.float32),
      pltpu.VMEM_SHARED((np_, f), jnp.float32),
      pltpu.SemaphoreType.DMA,
      pltpu.SemaphoreType.DMA,
  ]
  if with_deg:
    out_type.append(jax.ShapeDtypeStruct((NC, np_, f), jnp.float32))
    scratch += [
        pltpu.VMEM((CH, f), jnp.float32),
        pltpu.VMEM_SHARED((np_, f), jnp.float32),
    ]

  def body(*refs):
    if with_deg:
      (feat, srcp, dstp, zf, ones_h,
       agg_out, deg_out,
       idx_v, rows_v, acc_sh, gsem, ssem, ones_v, deg_sh) = refs
    else:
      (feat, srcp, dstp, zf,
       agg_out,
       idx_v, rows_v, acc_sh, gsem, ssem) = refs

    cid = lax.axis_index("c")
    sid = lax.axis_index("s")
    wid = cid * NS + sid
    r0 = sid * rpt

    @pl.when(sid < kz)
    def _zero():
      pltpu.sync_copy(zf, acc_sh.at[pl.ds(r0, rpt), :])
      if with_deg:
        pltpu.sync_copy(zf, deg_sh.at[pl.ds(r0, rpt), :])

    if with_deg:
      pltpu.sync_copy(ones_h, ones_v)
    plsc.subcore_barrier()

    c0 = wid * cpt

    def idx_copy(s, b):
      pltpu.sync_copy(srcp.at[pl.ds((c0 + s) * CH, CH)], idx_v.at[b, 0])
      pltpu.sync_copy(dstp.at[pl.ds((c0 + s) * CH, CH)], idx_v.at[b, 1])

    def fire_gather(b):
      pltpu.async_copy(feat.at[idx_v.at[b, 0]], rows_v.at[b], gsem)

    def wait_gather():
      pltpu.make_async_copy(feat.at[pl.ds(0, CH), :],
                            rows_v.at[0], gsem).wait()

    def fire_scatter(b):
      pltpu.async_copy(rows_v.at[b], acc_sh.at[idx_v.at[b, 1]],
                       ssem, add=True)
      if with_deg:
        pltpu.async_copy(ones_v, deg_sh.at[idx_v.at[b, 1]], ssem, add=True)

    def wait_scatter():
      for _ in range(2 if with_deg else 1):
        pltpu.make_async_copy(feat.at[pl.ds(0, CH), :],
                              rows_v.at[0], ssem).wait()

    idx_copy(0, 0)
    fire_gather(0)
    for s in range(cpt):
      b = s % 2
      wait_gather()
      if s >= 1:
        wait_scatter()
      if s + 1 < cpt:
        idx_copy(s + 1, 1 - b)
        fire_gather(1 - b)
      fire_scatter(b)
    wait_scatter()
    plsc.subcore_barrier()

    @pl.when(sid < kz)
    def _writeout():
      pltpu.sync_copy(acc_sh.at[pl.ds(r0, rpt), :],
                      agg_out.at[cid, pl.ds(r0, rpt), :])
      if with_deg:
        pltpu.sync_copy(deg_sh.at[pl.ds(r0, rpt), :],
                        deg_out.at[cid, pl.ds(r0, rpt), :])

  return pl.kernel(body, out_type=tuple(out_type), mesh=_mesh(),
                   scratch_types=tuple(scratch), interpret=interpret,
                   compiler_params=_sc_params())


def _make_score(np_, ep, interpret=False):
  cpt = ep // (NW * CH)
  out_type = jax.ShapeDtypeStruct((ep,), jnp.float32)
  scratch = (
      pltpu.VMEM((np_,), jnp.float32),
      pltpu.VMEM((np_,), jnp.float32),
      pltpu.VMEM((2, CH), jnp.int32),
      pltpu.VMEM((CH,), jnp.float32),
  )

  def body(s0_h, s1_h, srcp, dstp, out_h, s0_v, s1_v, idx_v, out_v):
    cid = lax.axis_index("c")
    sid = lax.axis_index("s")
    wid = cid * NS + sid
    pltpu.sync_copy(s0_h, s0_v)
    pltpu.sync_copy(s1_h, s1_v)
    c0 = wid * cpt

    def chunk(c, carry):
      pltpu.sync_copy(srcp.at[pl.ds(c * CH, CH)], idx_v.at[0])
      pltpu.sync_copy(dstp.at[pl.ds(c * CH, CH)], idx_v.at[1])

      def grp(i, carry2):
        sv = idx_v[0, pl.ds(i * 16, 16)]
        dv = idx_v[1, pl.ds(i * 16, 16)]
        out_v[pl.ds(i * 16, 16)] = (plsc.load_gather(s0_v, [sv])
                                    + plsc.load_gather(s1_v, [dv]))
        return carry2

      lax.fori_loop(0, CH // 16, grp, 0)
      pltpu.sync_copy(out_v, out_h.at[pl.ds(c * CH, CH)])
      return carry

    lax.fori_loop(c0, c0 + cpt, chunk, 0)

  return pl.kernel(body, out_type=out_type, mesh=_mesh(),
                   scratch_types=scratch, interpret=interpret,
                   compiler_params=_sc_params())


def _tc_project(h, w1):
  n, d = h.shape
  f = w1.shape[1]
  bn = 2000

  def body(h_ref, w_ref, o_ref):
    o_ref[...] = jnp.dot(h_ref[...], w_ref[...],
                         preferred_element_type=jnp.float32,
                         precision=jax.lax.Precision.HIGHEST)

  return pl.pallas_call(
      body,
      grid=(n // bn,),
      in_specs=[pl.BlockSpec((bn, d), lambda i: (i, 0)),
                pl.BlockSpec((d, f), lambda i: (0, 0))],
      out_specs=pl.BlockSpec((bn, f), lambda i: (i, 0)),
      out_shape=jax.ShapeDtypeStruct((n, f), jnp.float32),
  )(h, w1)


def _tc_layer1(aggp, degp, p1, b1):
  n, f = p1.shape

  def body(aggp_ref, degp_ref, p1_ref, b1_ref, hh_ref, inv_ref):
    deg = degp_ref[0, :n, 0:1] + degp_ref[1, :n, 0:1]
    inv = 1.0 / (deg + 1.0)
    hn = (aggp_ref[0, :n, :] + aggp_ref[1, :n, :] + p1_ref[...]) * inv
    hh_ref[...] = jnp.maximum(hn + b1_ref[...], 0.0)
    inv_ref[...] = inv

  return pl.pallas_call(
      body,
      out_shape=(jax.ShapeDtypeStruct((n, f), jnp.float32),
                 jax.ShapeDtypeStruct((n, 1), jnp.float32)),
  )(aggp, degp, p1, b1)


def _tc_layer2(agg2p, hh, inv, w2, wm, b2, bm):
  n, f = hh.shape
  d = w2.shape[1]

  def body(agg2p_ref, hh_ref, inv_ref, w2_ref, wm_ref, b2_ref, bm_ref,
           s0_ref, s1_ref):
    hn2 = (agg2p_ref[0, :n, :] + agg2p_ref[1, :n, :] + hh_ref[...]) \
        * inv_ref[...]
    wm_top = wm_ref[:d, :]
    wm_bot = wm_ref[d:, :]
    hi = jax.lax.Precision.HIGHEST
    u0 = jnp.dot(w2_ref[...], wm_top, preferred_element_type=jnp.float32,
                 precision=hi)
    u1 = jnp.dot(w2_ref[...], wm_bot, preferred_element_type=jnp.float32,
                 precision=hi)
    c0 = jnp.dot(b2_ref[...], wm_top, preferred_element_type=jnp.float32,
                 precision=hi)
    c1 = jnp.dot(b2_ref[...], wm_bot, preferred_element_type=jnp.float32,
                 precision=hi)
    s0_ref[...] = jnp.dot(hn2, u0, preferred_element_type=jnp.float32,
                          precision=hi) + c0[0, 0] + bm_ref[0, 0]
    s1_ref[...] = jnp.dot(hn2, u1, preferred_element_type=jnp.float32,
                          precision=hi) + c1[0, 0]

  return pl.pallas_call(
      body,
      out_shape=(jax.ShapeDtypeStruct((n, 1), jnp.float32),
                 jax.ShapeDtypeStruct((n, 1), jnp.float32)),
  )(agg2p, hh, inv, w2, wm, b2, bm)


@jax.jit
def kernel(h, edge_index, W1, b1, W2, b2, Wm, bm):
  n, d = h.shape
  e = edge_index.shape[1]
  f = W1.shape[1]
  np_ = n + NPAD
  step = NW * CH
  ep = ((e + step - 1) // step) * step

  srcp = jnp.concatenate([edge_index[0], jnp.zeros((ep - e,), jnp.int32)])
  dstp = jnp.concatenate([edge_index[1], jnp.full((ep - e,), n, jnp.int32)])
  _, rpt = _slice_split(np_)
  zf = jnp.zeros((rpt, f), jnp.float32)
  ones = jnp.ones((CH, f), jnp.float32)

  p1 = _tc_project(h, W1)
  aggp, degp = _make_scatter(n, np_, f, ep, True)(p1, srcp, dstp, zf, ones)
  hh, inv = _tc_layer1(aggp, degp, p1, b1.reshape(1, f))
  (agg2p,) = _make_scatter(n, np_, f, ep, False)(hh, srcp, dstp, zf)
  s0, s1 = _tc_layer2(agg2p, hh, inv, W2, Wm,
                      b2.reshape(1, d), bm.reshape(1, 1))
  s0p = jnp.pad(s0.reshape(n), (0, NPAD))
  s1p = jnp.pad(s1.reshape(n), (0, NPAD))
  score = _make_score(np_, ep)(s0p, s1p, srcp, dstp)
  return score[:e].reshape(e, 1)


# speedup vs baseline: 15.9110x; 1.0027x over previous
import jax
import jax.numpy as jnp
from jax import lax
from jax.experimental import pallas as pl
from jax.experimental.pallas import tpu as pltpu
from jax.experimental.pallas import tpu_sc as plsc

NC = 2
NS = 16
NW = NC * NS
CH = 2048
NPAD = 240


def _mesh():
  return plsc.VectorSubcoreMesh(
      core_axis_name="c", subcore_axis_name="s",
      num_cores=NC, num_subcores=NS)


def _sc_params():
  return pltpu.CompilerParams(
      use_tc_tiling_on_sc=False, needs_layout_passes=False)


def _slice_split(n):
  for k in range(NS, 0, -1):
    if n % k == 0 and (n // k) % 8 == 0:
      return k, n // k
  raise ValueError(n)


def _make_scatter(n, np_, f, ep, with_deg, interpret=False):
  kz, rpt = _slice_split(np_)
  cpt = ep // (NW * CH)

  out_type = [jax.ShapeDtypeStruct((NC, np_, f), jnp.float32)]
  scratch = [
      pltpu.VMEM((2, 2, CH), jnp.int32),
      pltpu.VMEM((2, CH, f), jnp.float32),
      pltpu.VMEM_SHARED((np_, f), jnp.float32),
      pltpu.SemaphoreType.DMA,
      pltpu.SemaphoreType.DMA,
  ]
  if with_deg:
    out_type.append(jax.ShapeDtypeStruct((NC, np_, f), jnp.float32))
    scratch += [
        pltpu.VMEM((CH, f), jnp.float32),
        pltpu.VMEM_SHARED((np_, f), jnp.float32),
    ]

  def body(*refs):
    if with_deg:
      (feat, srcp, dstp, zf, ones_h,
       agg_out, deg_out,
       idx_v, rows_v, acc_sh, gsem, ssem, ones_v, deg_sh) = refs
    else:
      (feat, srcp, dstp, zf,
       agg_out,
       idx_v, rows_v, acc_sh, gsem, ssem) = refs

    cid = lax.axis_index("c")
    sid = lax.axis_index("s")
    wid = cid * NS + sid
    r0 = sid * rpt

    @pl.when(sid < kz)
    def _zero():
      pltpu.sync_copy(zf, acc_sh.at[pl.ds(r0, rpt), :])
      if with_deg:
        pltpu.sync_copy(zf, deg_sh.at[pl.ds(r0, rpt), :])

    if with_deg:
      pltpu.sync_copy(ones_h, ones_v)
    plsc.subcore_barrier()

    c0 = wid * cpt

    def idx_copy(s, b):
      pltpu.sync_copy(srcp.at[pl.ds((c0 + s) * CH, CH)], idx_v.at[b, 0])
      pltpu.sync_copy(dstp.at[pl.ds((c0 + s) * CH, CH)], idx_v.at[b, 1])

    def fire_gather(b):
      pltpu.async_copy(feat.at[idx_v.at[b, 0]], rows_v.at[b], gsem)

    def wait_gather():
      pltpu.make_async_copy(feat.at[pl.ds(0, CH), :],
                            rows_v.at[0], gsem).wait()

    def fire_scatter(b):
      pltpu.async_copy(rows_v.at[b], acc_sh.at[idx_v.at[b, 1]],
                       ssem, add=True)
      if with_deg:
        pltpu.async_copy(ones_v, deg_sh.at[idx_v.at[b, 1]], ssem, add=True)

    def wait_scatter():
      for _ in range(2 if with_deg else 1):
        pltpu.make_async_copy(feat.at[pl.ds(0, CH), :],
                              rows_v.at[0], ssem).wait()

    idx_copy(0, 0)
    fire_gather(0)
    for s in range(cpt):
      b = s % 2
      wait_gather()
      if s >= 1:
        wait_scatter()
      if s + 1 < cpt:
        idx_copy(s + 1, 1 - b)
        fire_gather(1 - b)
      fire_scatter(b)
    wait_scatter()
    plsc.subcore_barrier()

    @pl.when(sid < kz)
    def _writeout():
      pltpu.sync_copy(acc_sh.at[pl.ds(r0, rpt), :],
                      agg_out.at[cid, pl.ds(r0, rpt), :])
      if with_deg:
        pltpu.sync_copy(deg_sh.at[pl.ds(r0, rpt), :],
                        deg_out.at[cid, pl.ds(r0, rpt), :])

  return pl.kernel(body, out_type=tuple(out_type), mesh=_mesh(),
                   scratch_types=tuple(scratch), interpret=interpret,
                   compiler_params=_sc_params())


def _make_score(np_, ep, interpret=False):
  cpt = ep // (NW * CH)
  out_type = jax.ShapeDtypeStruct((ep,), jnp.float32)
  scratch = (
      pltpu.VMEM((np_,), jnp.float32),
      pltpu.VMEM((np_,), jnp.float32),
      pltpu.VMEM((2, CH), jnp.int32),
      pltpu.VMEM((CH,), jnp.float32),
  )

  def body(s0_h, s1_h, srcp, dstp, out_h, s0_v, s1_v, idx_v, out_v):
    cid = lax.axis_index("c")
    sid = lax.axis_index("s")
    wid = cid * NS + sid
    pltpu.sync_copy(s0_h, s0_v)
    pltpu.sync_copy(s1_h, s1_v)
    c0 = wid * cpt

    def chunk(c, carry):
      pltpu.sync_copy(srcp.at[pl.ds(c * CH, CH)], idx_v.at[0])
      pltpu.sync_copy(dstp.at[pl.ds(c * CH, CH)], idx_v.at[1])

      def grp(i, carry2):
        sv = idx_v[0, pl.ds(i * 16, 16)]
        dv = idx_v[1, pl.ds(i * 16, 16)]
        out_v[pl.ds(i * 16, 16)] = (plsc.load_gather(s0_v, [sv])
                                    + plsc.load_gather(s1_v, [dv]))
        return carry2

      lax.fori_loop(0, CH // 16, grp, 0)
      pltpu.sync_copy(out_v, out_h.at[pl.ds(c * CH, CH)])
      return carry

    lax.fori_loop(c0, c0 + cpt, chunk, 0)

  return pl.kernel(body, out_type=out_type, mesh=_mesh(),
                   scratch_types=scratch, interpret=interpret,
                   compiler_params=_sc_params())


def _tc_project(h, w1):
  n, d = h.shape
  f = w1.shape[1]
  bn = 2000

  def body(h_ref, w_ref, o_ref):
    o_ref[...] = jnp.dot(h_ref[...], w_ref[...],
                         preferred_element_type=jnp.float32,
                         precision=jax.lax.Precision.HIGHEST)

  return pl.pallas_call(
      body,
      grid=(n // bn,),
      in_specs=[pl.BlockSpec((bn, d), lambda i: (i, 0)),
                pl.BlockSpec((d, f), lambda i: (0, 0))],
      out_specs=pl.BlockSpec((bn, f), lambda i: (i, 0)),
      out_shape=jax.ShapeDtypeStruct((n, f), jnp.float32),
  )(h, w1)


def _tc_layer1(aggp, degp, p1, b1):
  n, f = p1.shape

  def body(aggp_ref, degp_ref, p1_ref, b1_ref, hh_ref, inv_ref):
    deg = degp_ref[0, :n, 0:1] + degp_ref[1, :n, 0:1]
    inv = 1.0 / (deg + 1.0)
    hn = (aggp_ref[0, :n, :] + aggp_ref[1, :n, :] + p1_ref[...]) * inv
    hh_ref[...] = jnp.maximum(hn + b1_ref[...], 0.0)
    inv_ref[...] = inv

  return pl.pallas_call(
      body,
      out_shape=(jax.ShapeDtypeStruct((n, f), jnp.float32),
                 jax.ShapeDtypeStruct((n, 1), jnp.float32)),
  )(aggp, degp, p1, b1)


def _tc_layer2(agg2p, hh, inv, w2, wm, b2, bm):
  n, f = hh.shape
  d = w2.shape[1]

  def body(agg2p_ref, hh_ref, inv_ref, w2_ref, wm_ref, b2_ref, bm_ref,
           s0_ref, s1_ref):
    hn2 = (agg2p_ref[0, :n, :] + agg2p_ref[1, :n, :] + hh_ref[...]) \
        * inv_ref[...]
    wm_top = wm_ref[:d, :]
    wm_bot = wm_ref[d:, :]
    hi = jax.lax.Precision.HIGHEST
    u0 = jnp.dot(w2_ref[...], wm_top, preferred_element_type=jnp.float32,
                 precision=hi)
    u1 = jnp.dot(w2_ref[...], wm_bot, preferred_element_type=jnp.float32,
                 precision=hi)
    c0 = jnp.dot(b2_ref[...], wm_top, preferred_element_type=jnp.float32,
                 precision=hi)
    c1 = jnp.dot(b2_ref[...], wm_bot, preferred_element_type=jnp.float32,
                 precision=hi)
    s0_ref[...] = jnp.dot(hn2, u0, preferred_element_type=jnp.float32,
                          precision=hi) + c0[0, 0] + bm_ref[0, 0]
    s1_ref[...] = jnp.dot(hn2, u1, preferred_element_type=jnp.float32,
                          precision=hi) + c1[0, 0]

  return pl.pallas_call(
      body,
      out_shape=(jax.ShapeDtypeStruct((n, 1), jnp.float32),
                 jax.ShapeDtypeStruct((n, 1), jnp.float32)),
  )(agg2p, hh, inv, w2, wm, b2, bm)


@jax.jit
def kernel(h, edge_index, W1, b1, W2, b2, Wm, bm):
  n, d = h.shape
  e = edge_index.shape[1]
  f = W1.shape[1]
  np_ = n + NPAD
  step = NW * CH
  ep = ((e + step - 1) // step) * step

  pad_dst = n + (jnp.arange(ep - e, dtype=jnp.int32) % NPAD)
  srcp = jnp.concatenate([edge_index[0], jnp.zeros((ep - e,), jnp.int32)])
  dstp = jnp.concatenate([edge_index[1], pad_dst])
  _, rpt = _slice_split(np_)
  zf = jnp.zeros((rpt, f), jnp.float32)
  ones = jnp.ones((CH, f), jnp.float32)

  p1 = _tc_project(h, W1)
  aggp, degp = _make_scatter(n, np_, f, ep, True)(p1, srcp, dstp, zf, ones)
  hh, inv = _tc_layer1(aggp, degp, p1, b1.reshape(1, f))
  (agg2p,) = _make_scatter(n, np_, f, ep, False)(hh, srcp, dstp, zf)
  s0, s1 = _tc_layer2(agg2p, hh, inv, W2, Wm,
                      b2.reshape(1, d), bm.reshape(1, 1))
  s0p = jnp.pad(s0.reshape(n), (0, NPAD))
  s1p = jnp.pad(s1.reshape(n), (0, NPAD))
  score = _make_score(np_, ep)(s0p, s1p, srcp, dstp)
  return score[:e].reshape(e, 1)


# speedup vs baseline: 15.9934x; 1.0052x over previous
import jax
import jax.numpy as jnp
from jax import lax
from jax.experimental import pallas as pl
from jax.experimental.pallas import tpu as pltpu
from jax.experimental.pallas import tpu_sc as plsc

NC = 2
NS = 16
NW = NC * NS
CH = 2048
NPAD = 240


def _mesh():
  return plsc.VectorSubcoreMesh(
      core_axis_name="c", subcore_axis_name="s",
      num_cores=NC, num_subcores=NS)


def _sc_params():
  return pltpu.CompilerParams(
      use_tc_tiling_on_sc=False, needs_layout_passes=False)


def _slice_split(n):
  for k in range(NS, 0, -1):
    if n % k == 0 and (n // k) % 8 == 0:
      return k, n // k
  raise ValueError(n)


def _make_scatter(n, np_, f, ep, with_deg, interpret=False):
  kz, rpt = _slice_split(np_)
  cpt = ep // (NW * CH)

  out_type = [jax.ShapeDtypeStruct((NC, np_, f), jnp.float32)]
  scratch = [
      pltpu.VMEM((2, 2, CH), jnp.int32),
      pltpu.VMEM((2, CH, f), jnp.float32),
      pltpu.VMEM_SHARED((np_, f), jnp.float32),
      pltpu.SemaphoreType.DMA,
      pltpu.SemaphoreType.DMA,
  ]
  if with_deg:
    out_type.append(jax.ShapeDtypeStruct((NC, np_, f), jnp.float32))
    scratch += [
        pltpu.VMEM((CH, f), jnp.float32),
        pltpu.VMEM_SHARED((np_, f), jnp.float32),
    ]

  def body(*refs):
    if with_deg:
      (feat, srcp, dstp, zf, ones_h,
       agg_out, deg_out,
       idx_v, rows_v, acc_sh, gsem, ssem, ones_v, deg_sh) = refs
    else:
      (feat, srcp, dstp, zf,
       agg_out,
       idx_v, rows_v, acc_sh, gsem, ssem) = refs

    cid = lax.axis_index("c")
    sid = lax.axis_index("s")
    wid = cid * NS + sid
    r0 = sid * rpt

    @pl.when(sid < kz)
    def _zero():
      pltpu.sync_copy(zf, acc_sh.at[pl.ds(r0, rpt), :])
      if with_deg:
        pltpu.sync_copy(zf, deg_sh.at[pl.ds(r0, rpt), :])

    if with_deg:
      pltpu.sync_copy(ones_h, ones_v)
    plsc.subcore_barrier()

    c0 = wid * cpt

    def idx_copy(s, b):
      pltpu.sync_copy(srcp.at[pl.ds((c0 + s) * CH, CH)], idx_v.at[b, 0])
      pltpu.sync_copy(dstp.at[pl.ds((c0 + s) * CH, CH)], idx_v.at[b, 1])

    def fire_gather(b):
      pltpu.async_copy(feat.at[idx_v.at[b, 0]], rows_v.at[b], gsem)

    def wait_gather():
      pltpu.make_async_copy(feat.at[pl.ds(0, CH), :],
                            rows_v.at[0], gsem).wait()

    def fire_scatter(b):
      pltpu.async_copy(rows_v.at[b], acc_sh.at[idx_v.at[b, 1]],
                       ssem, add=True)
      if with_deg:
        pltpu.async_copy(ones_v, deg_sh.at[idx_v.at[b, 1]], ssem, add=True)

    def wait_scatter():
      for _ in range(2 if with_deg else 1):
        pltpu.make_async_copy(feat.at[pl.ds(0, CH), :],
                              rows_v.at[0], ssem).wait()

    idx_copy(0, 0)
    fire_gather(0)
    for s in range(cpt):
      b = s % 2
      wait_gather()
      if s >= 1:
        wait_scatter()
      if s + 1 < cpt:
        idx_copy(s + 1, 1 - b)
        fire_gather(1 - b)
      fire_scatter(b)
    wait_scatter()
    plsc.subcore_barrier()

    @pl.when(sid < kz)
    def _writeout():
      pltpu.sync_copy(acc_sh.at[pl.ds(r0, rpt), :],
                      agg_out.at[cid, pl.ds(r0, rpt), :])
      if with_deg:
        pltpu.sync_copy(deg_sh.at[pl.ds(r0, rpt), :],
                        deg_out.at[cid, pl.ds(r0, rpt), :])

  return pl.kernel(body, out_type=tuple(out_type), mesh=_mesh(),
                   scratch_types=tuple(scratch), interpret=interpret,
                   compiler_params=_sc_params())


def _make_score(np_, ep, interpret=False):
  cpt = ep // (NW * CH)
  out_type = jax.ShapeDtypeStruct((ep,), jnp.float32)
  scratch = (
      pltpu.VMEM((np_,), jnp.float32),
      pltpu.VMEM((np_,), jnp.float32),
      pltpu.VMEM((2, CH), jnp.int32),
      pltpu.VMEM((CH,), jnp.float32),
  )

  def body(s0_h, s1_h, srcp, dstp, out_h, s0_v, s1_v, idx_v, out_v):
    cid = lax.axis_index("c")
    sid = lax.axis_index("s")
    wid = cid * NS + sid
    pltpu.sync_copy(s0_h, s0_v)
    pltpu.sync_copy(s1_h, s1_v)
    c0 = wid * cpt

    def chunk(c, carry):
      pltpu.sync_copy(srcp.at[pl.ds(c * CH, CH)], idx_v.at[0])
      pltpu.sync_copy(dstp.at[pl.ds(c * CH, CH)], idx_v.at[1])

      def grp(i, carry2):
        sv = idx_v[0, pl.ds(i * 16, 16)]
        dv = idx_v[1, pl.ds(i * 16, 16)]
        out_v[pl.ds(i * 16, 16)] = (plsc.load_gather(s0_v, [sv])
                                    + plsc.load_gather(s1_v, [dv]))
        return carry2

      lax.fori_loop(0, CH // 16, grp, 0)
      pltpu.sync_copy(out_v, out_h.at[pl.ds(c * CH, CH)])
      return carry

    lax.fori_loop(c0, c0 + cpt, chunk, 0)

  return pl.kernel(body, out_type=out_type, mesh=_mesh(),
                   scratch_types=scratch, interpret=interpret,
                   compiler_params=_sc_params())


def _tc_project(h, w1):
  n, d = h.shape
  f = w1.shape[1]
  bn = 2000

  def body(h_ref, w_ref, o_ref):
    o_ref[...] = jnp.dot(h_ref[...], w_ref[...],
                         preferred_element_type=jnp.float32,
                         precision=jax.lax.Precision.HIGHEST)

  return pl.pallas_call(
      body,
      grid=(n // bn,),
      in_specs=[pl.BlockSpec((bn, d), lambda i: (i, 0)),
                pl.BlockSpec((d, f), lambda i: (0, 0))],
      out_specs=pl.BlockSpec((bn, f), lambda i: (i, 0)),
      out_shape=jax.ShapeDtypeStruct((n, f), jnp.float32),
  )(h, w1)


def _tc_layer1(aggp, degp, p1, b1):
  n, f = p1.shape

  def body(aggp_ref, degp_ref, p1_ref, b1_ref, hh_ref):
    deg = degp_ref[0, :n, 0:1] + degp_ref[1, :n, 0:1]
    inv = 1.0 / (deg + 1.0)
    hn = (aggp_ref[0, :n, :] + aggp_ref[1, :n, :] + p1_ref[...]) * inv
    hh_ref[...] = jnp.maximum(hn + b1_ref[...], 0.0)

  return pl.pallas_call(
      body,
      out_shape=jax.ShapeDtypeStruct((n, f), jnp.float32),
  )(aggp, degp, p1, b1)


def _tc_layer2(agg2p, degp, hh, w2, wm, b2, bm):
  n, f = hh.shape
  d = w2.shape[1]

  def body(agg2p_ref, degp_ref, hh_ref, w2_ref, wm_ref, b2_ref, bm_ref,
           s0_ref, s1_ref):
    deg = degp_ref[0, :n, 0:1] + degp_ref[1, :n, 0:1]
    inv = 1.0 / (deg + 1.0)
    hn2 = (agg2p_ref[0, :n, :] + agg2p_ref[1, :n, :] + hh_ref[...]) * inv
    wm_top = wm_ref[:d, :]
    wm_bot = wm_ref[d:, :]
    hi = jax.lax.Precision.HIGHEST
    u0 = jnp.dot(w2_ref[...], wm_top, preferred_element_type=jnp.float32,
                 precision=hi)
    u1 = jnp.dot(w2_ref[...], wm_bot, preferred_element_type=jnp.float32,
                 precision=hi)
    c0 = jnp.dot(b2_ref[...], wm_top, preferred_element_type=jnp.float32,
                 precision=hi)
    c1 = jnp.dot(b2_ref[...], wm_bot, preferred_element_type=jnp.float32,
                 precision=hi)
    s0_ref[...] = jnp.dot(hn2, u0, preferred_element_type=jnp.float32,
                          precision=hi)[:, 0] + c0[0, 0] + bm_ref[0, 0]
    s1_ref[...] = jnp.dot(hn2, u1, preferred_element_type=jnp.float32,
                          precision=hi)[:, 0] + c1[0, 0]

  return pl.pallas_call(
      body,
      out_shape=(jax.ShapeDtypeStruct((n,), jnp.float32),
                 jax.ShapeDtypeStruct((n,), jnp.float32)),
  )(agg2p, degp, hh, w2, wm, b2, bm)


@jax.jit
def kernel(h, edge_index, W1, b1, W2, b2, Wm, bm):
  n, d = h.shape
  e = edge_index.shape[1]
  f = W1.shape[1]
  np_ = n + NPAD
  step = NW * CH
  ep = ((e + step - 1) // step) * step

  pad_dst = n + (jnp.arange(ep - e, dtype=jnp.int32) % NPAD)
  srcp = jnp.concatenate([edge_index[0], jnp.zeros((ep - e,), jnp.int32)])
  dstp = jnp.concatenate([edge_index[1], pad_dst])
  _, rpt = _slice_split(np_)
  zf = jnp.zeros((rpt, f), jnp.float32)
  ones = jnp.ones((CH, f), jnp.float32)

  p1 = _tc_project(h, W1)
  aggp, degp = _make_scatter(n, np_, f, ep, True)(p1, srcp, dstp, zf, ones)
  hh = _tc_layer1(aggp, degp, p1, b1.reshape(1, f))
  (agg2p,) = _make_scatter(n, np_, f, ep, False)(hh, srcp, dstp, zf)
  s0, s1 = _tc_layer2(agg2p, degp, hh, W2, Wm,
                      b2.reshape(1, d), bm.reshape(1, 1))
  s0p = jnp.pad(s0, (0, NPAD))
  s1p = jnp.pad(s1, (0, NPAD))
  score = _make_score(np_, ep)(s0p, s1p, srcp, dstp)
  return score[:e].reshape(e, 1)


# speedup vs baseline: 16.7711x; 1.0486x over previous
import jax
import jax.numpy as jnp
import numpy as np
from jax import lax
from jax.experimental import pallas as pl
from jax.experimental.pallas import tpu as pltpu
from jax.experimental.pallas import tpu_sc as plsc

NC = 2
NS = 16
NW = NC * NS
CH = 2048
NPAD = 240


def _mesh():
  return plsc.VectorSubcoreMesh(
      core_axis_name="c", subcore_axis_name="s",
      num_cores=NC, num_subcores=NS)


def _sc_params():
  return pltpu.CompilerParams(
      use_tc_tiling_on_sc=False, needs_layout_passes=False)


def _slice_split(n):
  for k in range(NS, 0, -1):
    if n % k == 0 and (n // k) % 8 == 0:
      return k, n // k
  raise ValueError(n)


def _make_scatter(n, np_, f, ep, with_deg, interpret=False):
  kz, rpt = _slice_split(np_)
  cpt = ep // (NW * CH)

  out_type = [jax.ShapeDtypeStruct((NC, np_, f), jnp.float32)]
  scratch = [
      pltpu.VMEM((2, 2, CH), jnp.int32),
      pltpu.VMEM((2, CH, f), jnp.float32),
      pltpu.VMEM_SHARED((np_, f), jnp.float32),
      pltpu.SemaphoreType.DMA,
      pltpu.SemaphoreType.DMA,
  ]
  if with_deg:
    out_type.append(jax.ShapeDtypeStruct((NC, np_, f), jnp.float32))
    scratch += [
        pltpu.VMEM((CH, f), jnp.float32),
        pltpu.VMEM_SHARED((np_, f), jnp.float32),
    ]

  def body(*refs):
    if with_deg:
      (feat, srcp, dstp, zf, ones_h,
       agg_out, deg_out,
       idx_v, rows_v, acc_sh, gsem, ssem, ones_v, deg_sh) = refs
    else:
      (feat, srcp, dstp, zf,
       agg_out,
       idx_v, rows_v, acc_sh, gsem, ssem) = refs

    cid = lax.axis_index("c")
    sid = lax.axis_index("s")
    wid = cid * NS + sid
    r0 = sid * rpt

    @pl.when(sid < kz)
    def _zero():
      pltpu.sync_copy(zf, acc_sh.at[pl.ds(r0, rpt), :])
      if with_deg:
        pltpu.sync_copy(zf, deg_sh.at[pl.ds(r0, rpt), :])

    if with_deg:
      pltpu.sync_copy(ones_h, ones_v)
    plsc.subcore_barrier()

    c0 = wid * cpt

    def idx_copy(s, b):
      pltpu.sync_copy(srcp.at[pl.ds((c0 + s) * CH, CH)], idx_v.at[b, 0])
      pltpu.sync_copy(dstp.at[pl.ds((c0 + s) * CH, CH)], idx_v.at[b, 1])

    def fire_gather(b):
      pltpu.async_copy(feat.at[idx_v.at[b, 0]], rows_v.at[b], gsem)

    def wait_gather():
      pltpu.make_async_copy(feat.at[pl.ds(0, CH), :],
                            rows_v.at[0], gsem).wait()

    def fire_scatter(b):
      pltpu.async_copy(rows_v.at[b], acc_sh.at[idx_v.at[b, 1]],
                       ssem, add=True)
      if with_deg:
        pltpu.async_copy(ones_v, deg_sh.at[idx_v.at[b, 1]], ssem, add=True)

    def wait_scatter():
      for _ in range(2 if with_deg else 1):
        pltpu.make_async_copy(feat.at[pl.ds(0, CH), :],
                              rows_v.at[0], ssem).wait()

    idx_copy(0, 0)
    fire_gather(0)
    for s in range(cpt):
      b = s % 2
      wait_gather()
      if s >= 1:
        wait_scatter()
      if s + 1 < cpt:
        idx_copy(s + 1, 1 - b)
        fire_gather(1 - b)
      fire_scatter(b)
    wait_scatter()
    plsc.subcore_barrier()

    @pl.when(sid < kz)
    def _writeout():
      pltpu.sync_copy(acc_sh.at[pl.ds(r0, rpt), :],
                      agg_out.at[cid, pl.ds(r0, rpt), :])
      if with_deg:
        pltpu.sync_copy(deg_sh.at[pl.ds(r0, rpt), :],
                        deg_out.at[cid, pl.ds(r0, rpt), :])

  return pl.kernel(body, out_type=tuple(out_type), mesh=_mesh(),
                   scratch_types=tuple(scratch), interpret=interpret,
                   compiler_params=_sc_params())


def _make_score(np_, ep, interpret=False):
  cpt = ep // (NW * CH)
  out_type = jax.ShapeDtypeStruct((ep,), jnp.float32)
  scratch = (
      pltpu.VMEM((np_,), jnp.float32),
      pltpu.VMEM((np_,), jnp.float32),
      pltpu.VMEM((2, CH), jnp.int32),
      pltpu.VMEM((CH,), jnp.float32),
  )

  def body(s0_h, s1_h, srcp, dstp, out_h, s0_v, s1_v, idx_v, out_v):
    cid = lax.axis_index("c")
    sid = lax.axis_index("s")
    wid = cid * NS + sid
    pltpu.sync_copy(s0_h, s0_v)
    pltpu.sync_copy(s1_h, s1_v)
    c0 = wid * cpt

    def chunk(c, carry):
      pltpu.sync_copy(srcp.at[pl.ds(c * CH, CH)], idx_v.at[0])
      pltpu.sync_copy(dstp.at[pl.ds(c * CH, CH)], idx_v.at[1])

      def grp(i, carry2):
        sv = idx_v[0, pl.ds(i * 16, 16)]
        dv = idx_v[1, pl.ds(i * 16, 16)]
        out_v[pl.ds(i * 16, 16)] = (plsc.load_gather(s0_v, [sv])
                                    + plsc.load_gather(s1_v, [dv]))
        return carry2

      lax.fori_loop(0, CH // 16, grp, 0)
      pltpu.sync_copy(out_v, out_h.at[pl.ds(c * CH, CH)])
      return carry

    lax.fori_loop(c0, c0 + cpt, chunk, 0)

  return pl.kernel(body, out_type=out_type, mesh=_mesh(),
                   scratch_types=scratch, interpret=interpret,
                   compiler_params=_sc_params())


def _tc_project(h, w1):
  n, d = h.shape
  f = w1.shape[1]
  bn = 2000

  def body(h_ref, w_ref, o_ref):
    o_ref[...] = jnp.dot(h_ref[...], w_ref[...],
                         preferred_element_type=jnp.float32,
                         precision=jax.lax.Precision.HIGHEST)

  return pl.pallas_call(
      body,
      grid=(n // bn,),
      in_specs=[pl.BlockSpec((bn, d), lambda i: (i, 0)),
                pl.BlockSpec((d, f), lambda i: (0, 0))],
      out_specs=pl.BlockSpec((bn, f), lambda i: (i, 0)),
      out_shape=jax.ShapeDtypeStruct((n, f), jnp.float32),
  )(h, w1)


def _tc_layer1(aggp, degp, p1, b1):
  n, f = p1.shape

  def body(aggp_ref, degp_ref, p1_ref, b1_ref, hh_ref):
    deg = degp_ref[0, :n, 0:1] + degp_ref[1, :n, 0:1]
    inv = 1.0 / (deg + 1.0)
    hn = (aggp_ref[0, :n, :] + aggp_ref[1, :n, :] + p1_ref[...]) * inv
    hh_ref[...] = jnp.maximum(hn + b1_ref[...], 0.0)

  return pl.pallas_call(
      body,
      out_shape=jax.ShapeDtypeStruct((n, f), jnp.float32),
  )(aggp, degp, p1, b1)


def _tc_layer2(agg2p, degp, hh, w2, wm, b2, bm):
  n, f = hh.shape
  d = w2.shape[1]

  def body(agg2p_ref, degp_ref, hh_ref, w2_ref, wm_ref, b2_ref, bm_ref,
           s0_ref, s1_ref):
    deg = degp_ref[0, :n, 0:1] + degp_ref[1, :n, 0:1]
    inv = 1.0 / (deg + 1.0)
    hn2 = (agg2p_ref[0, :n, :] + agg2p_ref[1, :n, :] + hh_ref[...]) * inv
    wm_top = wm_ref[:d, :]
    wm_bot = wm_ref[d:, :]
    hi = jax.lax.Precision.HIGHEST
    u0 = jnp.dot(w2_ref[...], wm_top, preferred_element_type=jnp.float32,
                 precision=hi)
    u1 = jnp.dot(w2_ref[...], wm_bot, preferred_element_type=jnp.float32,
                 precision=hi)
    c0 = jnp.dot(b2_ref[...], wm_top, preferred_element_type=jnp.float32,
                 precision=hi)
    c1 = jnp.dot(b2_ref[...], wm_bot, preferred_element_type=jnp.float32,
                 precision=hi)
    zpad = jnp.zeros((NPAD,), jnp.float32)
    s0 = jnp.dot(hn2, u0, preferred_element_type=jnp.float32,
                 precision=hi)[:, 0] + c0[0, 0] + bm_ref[0, 0]
    s1 = jnp.dot(hn2, u1, preferred_element_type=jnp.float32,
                 precision=hi)[:, 0] + c1[0, 0]
    s0_ref[...] = jnp.concatenate([s0, zpad])
    s1_ref[...] = jnp.concatenate([s1, zpad])

  return pl.pallas_call(
      body,
      out_shape=(jax.ShapeDtypeStruct((n + NPAD,), jnp.float32),
                 jax.ShapeDtypeStruct((n + NPAD,), jnp.float32)),
  )(agg2p, degp, hh, w2, wm, b2, bm)


@jax.jit
def kernel(h, edge_index, W1, b1, W2, b2, Wm, bm):
  n, d = h.shape
  e = edge_index.shape[1]
  f = W1.shape[1]
  np_ = n + NPAD
  step = NW * CH
  ep = ((e + step - 1) // step) * step

  pad_block = np.stack([np.zeros((ep - e,), np.int32),
                        n + (np.arange(ep - e, dtype=np.int32) % NPAD)])
  eip = jnp.concatenate([edge_index, jnp.asarray(pad_block)], axis=1)
  srcp = eip[0]
  dstp = eip[1]
  _, rpt = _slice_split(np_)
  zf = jnp.asarray(np.zeros((rpt, f), np.float32))
  ones = jnp.asarray(np.ones((CH, f), np.float32))

  p1 = _tc_project(h, W1)
  aggp, degp = _make_scatter(n, np_, f, ep, True)(p1, srcp, dstp, zf, ones)
  hh = _tc_layer1(aggp, degp, p1, b1.reshape(1, f))
  (agg2p,) = _make_scatter(n, np_, f, ep, False)(hh, srcp, dstp, zf)
  s0p, s1p = _tc_layer2(agg2p, degp, hh, W2, Wm,
                        b2.reshape(1, d), bm.reshape(1, 1))
  score = _make_score(np_, ep)(s0p, s1p, srcp, dstp)
  return score[:e].reshape(e, 1)


# speedup vs baseline: 17.5494x; 1.0464x over previous
import jax
import jax.numpy as jnp
import numpy as np
from jax import lax
from jax.experimental import pallas as pl
from jax.experimental.pallas import tpu as pltpu
from jax.experimental.pallas import tpu_sc as plsc

NC = 2
NS = 16
NW = NC * NS
CH = 2048
NPAD = 240


def _mesh():
  return plsc.VectorSubcoreMesh(
      core_axis_name="c", subcore_axis_name="s",
      num_cores=NC, num_subcores=NS)


def _sc_params():
  return pltpu.CompilerParams(
      use_tc_tiling_on_sc=False, needs_layout_passes=False)


def _slice_split(n):
  for k in range(NS, 0, -1):
    if n % k == 0 and (n // k) % 8 == 0:
      return k, n // k
  raise ValueError(n)


def _make_scatter(n, np_, f, ep, with_deg, interpret=False):
  kz, rpt = _slice_split(np_)
  cpt = ep // (NW * CH)

  out_type = [jax.ShapeDtypeStruct((NC, np_, f), jnp.float32)]
  scratch = [
      pltpu.VMEM((2, 2, CH), jnp.int32),
      pltpu.VMEM((2, CH, f), jnp.float32),
      pltpu.VMEM_SHARED((np_, f), jnp.float32),
      pltpu.SemaphoreType.DMA,
      pltpu.SemaphoreType.DMA,
  ]
  if with_deg:
    out_type.append(jax.ShapeDtypeStruct((NC, np_, f), jnp.float32))
    scratch += [
        pltpu.VMEM((CH, f), jnp.float32),
        pltpu.VMEM_SHARED((np_, f), jnp.float32),
    ]

  def body(*refs):
    if with_deg:
      (feat, srcp, dstp, zf, ones_h,
       agg_out, deg_out,
       idx_v, rows_v, acc_sh, gsem, ssem, ones_v, deg_sh) = refs
    else:
      (feat, srcp, dstp, zf,
       agg_out,
       idx_v, rows_v, acc_sh, gsem, ssem) = refs

    cid = lax.axis_index("c")
    sid = lax.axis_index("s")
    wid = cid * NS + sid
    r0 = sid * rpt

    @pl.when(sid < kz)
    def _zero():
      pltpu.sync_copy(zf, acc_sh.at[pl.ds(r0, rpt), :])
      if with_deg:
        pltpu.sync_copy(zf, deg_sh.at[pl.ds(r0, rpt), :])

    if with_deg:
      pltpu.sync_copy(ones_h, ones_v)
    plsc.subcore_barrier()

    c0 = wid * cpt

    def idx_copy(s, b):
      pltpu.sync_copy(srcp.at[pl.ds((c0 + s) * CH, CH)], idx_v.at[b, 0])
      pltpu.sync_copy(dstp.at[pl.ds((c0 + s) * CH, CH)], idx_v.at[b, 1])

    def fire_gather(b):
      pltpu.async_copy(feat.at[idx_v.at[b, 0]], rows_v.at[b], gsem)

    def wait_gather():
      pltpu.make_async_copy(feat.at[pl.ds(0, CH), :],
                            rows_v.at[0], gsem).wait()

    def fire_scatter(b):
      pltpu.async_copy(rows_v.at[b], acc_sh.at[idx_v.at[b, 1]],
                       ssem, add=True)
      if with_deg:
        pltpu.async_copy(ones_v, deg_sh.at[idx_v.at[b, 1]], ssem, add=True)

    def wait_scatter():
      for _ in range(2 if with_deg else 1):
        pltpu.make_async_copy(feat.at[pl.ds(0, CH), :],
                              rows_v.at[0], ssem).wait()

    idx_copy(0, 0)
    fire_gather(0)
    for s in range(cpt):
      b = s % 2
      wait_gather()
      if s >= 1:
        wait_scatter()
      if s + 1 < cpt:
        idx_copy(s + 1, 1 - b)
        fire_gather(1 - b)
      fire_scatter(b)
    wait_scatter()
    plsc.subcore_barrier()

    @pl.when(sid < kz)
    def _writeout():
      pltpu.sync_copy(acc_sh.at[pl.ds(r0, rpt), :],
                      agg_out.at[cid, pl.ds(r0, rpt), :])
      if with_deg:
        pltpu.sync_copy(deg_sh.at[pl.ds(r0, rpt), :],
                        deg_out.at[cid, pl.ds(r0, rpt), :])

  return pl.kernel(body, out_type=tuple(out_type), mesh=_mesh(),
                   scratch_types=tuple(scratch), interpret=interpret,
                   compiler_params=_sc_params())


def _make_score(np_, ep, interpret=False):
  cpt = ep // (NW * CH)
  out_type = jax.ShapeDtypeStruct((ep,), jnp.float32)
  scratch = (
      pltpu.VMEM((np_,), jnp.float32),
      pltpu.VMEM((np_,), jnp.float32),
      pltpu.VMEM((2, CH), jnp.int32),
      pltpu.VMEM((CH,), jnp.float32),
  )

  def body(s0_h, s1_h, srcp, dstp, out_h, s0_v, s1_v, idx_v, out_v):
    cid = lax.axis_index("c")
    sid = lax.axis_index("s")
    wid = cid * NS + sid
    pltpu.sync_copy(s0_h, s0_v)
    pltpu.sync_copy(s1_h, s1_v)
    c0 = wid * cpt

    def chunk(c, carry):
      pltpu.sync_copy(srcp.at[pl.ds(c * CH, CH)], idx_v.at[0])
      pltpu.sync_copy(dstp.at[pl.ds(c * CH, CH)], idx_v.at[1])

      def grp(i, carry2):
        sv = idx_v[0, pl.ds(i * 16, 16)]
        dv = idx_v[1, pl.ds(i * 16, 16)]
        out_v[pl.ds(i * 16, 16)] = (plsc.load_gather(s0_v, [sv])
                                    + plsc.load_gather(s1_v, [dv]))
        return carry2

      lax.fori_loop(0, CH // 16, grp, 0)
      pltpu.sync_copy(out_v, out_h.at[pl.ds(c * CH, CH)])
      return carry

    lax.fori_loop(c0, c0 + cpt, chunk, 0)

  return pl.kernel(body, out_type=out_type, mesh=_mesh(),
                   scratch_types=scratch, interpret=interpret,
                   compiler_params=_sc_params())


def _tc_project(h, w1):
  n, d = h.shape
  f = w1.shape[1]
  bn = 2000

  def body(h_ref, w_ref, o_ref):
    o_ref[...] = jnp.dot(h_ref[...], w_ref[...],
                         preferred_element_type=jnp.float32,
                         precision=jax.lax.Precision.HIGHEST)

  return pl.pallas_call(
      body,
      grid=(n // bn,),
      in_specs=[pl.BlockSpec((bn, d), lambda i: (i, 0)),
                pl.BlockSpec((d, f), lambda i: (0, 0))],
      out_specs=pl.BlockSpec((bn, f), lambda i: (i, 0)),
      out_shape=jax.ShapeDtypeStruct((n, f), jnp.float32),
  )(h, w1)


def _tc_layer1(aggp, degp, p1, b1):
  n, f = p1.shape

  def body(aggp_ref, degp_ref, p1_ref, b1_ref, hh_ref):
    deg = degp_ref[0, :n, 0:1] + degp_ref[1, :n, 0:1]
    inv = 1.0 / (deg + 1.0)
    hn = (aggp_ref[0, :n, :] + aggp_ref[1, :n, :] + p1_ref[...]) * inv
    hh_ref[...] = jnp.maximum(hn + b1_ref[...], 0.0)

  return pl.pallas_call(
      body,
      out_shape=jax.ShapeDtypeStruct((n, f), jnp.float32),
  )(aggp, degp, p1, b1)


def _tc_layer2(agg2p, degp, hh, w2, wm, b2, bm):
  n, f = hh.shape
  d = w2.shape[1]

  def body(agg2p_ref, degp_ref, hh_ref, w2_ref, wm_ref, b2_ref, bm_ref,
           s0_ref, s1_ref):
    deg = degp_ref[0, :n, 0:1] + degp_ref[1, :n, 0:1]
    inv = 1.0 / (deg + 1.0)
    hn2 = (agg2p_ref[0, :n, :] + agg2p_ref[1, :n, :] + hh_ref[...]) * inv
    hh2 = jnp.dot(hn2, w2_ref[...],
                  preferred_element_type=jnp.float32) + b2_ref[...]
    zpad = jnp.zeros((NPAD,), jnp.float32)
    s0 = jnp.dot(hh2, wm_ref[:d, :],
                 preferred_element_type=jnp.float32)[:, 0] + bm_ref[0, 0]
    s1 = jnp.dot(hh2, wm_ref[d:, :],
                 preferred_element_type=jnp.float32)[:, 0]
    s0_ref[...] = jnp.concatenate([s0, zpad])
    s1_ref[...] = jnp.concatenate([s1, zpad])

  return pl.pallas_call(
      body,
      out_shape=(jax.ShapeDtypeStruct((n + NPAD,), jnp.float32),
                 jax.ShapeDtypeStruct((n + NPAD,), jnp.float32)),
  )(agg2p, degp, hh, w2, wm, b2, bm)


@jax.jit
def kernel(h, edge_index, W1, b1, W2, b2, Wm, bm):
  n, d = h.shape
  e = edge_index.shape[1]
  f = W1.shape[1]
  np_ = n + NPAD
  step = NW * CH
  ep = ((e + step - 1) // step) * step

  pad_block = np.stack([np.zeros((ep - e,), np.int32),
                        n + (np.arange(ep - e, dtype=np.int32) % NPAD)])
  eip = jnp.concatenate([edge_index, jnp.asarray(pad_block)], axis=1)
  srcp = eip[0]
  dstp = eip[1]
  _, rpt = _slice_split(np_)
  zf = jnp.asarray(np.zeros((rpt, f), np.float32))
  ones = jnp.asarray(np.ones((CH, f), np.float32))

  p1 = _tc_project(h, W1)
  aggp, degp = _make_scatter(n, np_, f, ep, True)(p1, srcp, dstp, zf, ones)
  hh = _tc_layer1(aggp, degp, p1, b1.reshape(1, f))
  (agg2p,) = _make_scatter(n, np_, f, ep, False)(hh, srcp, dstp, zf)
  s0p, s1p = _tc_layer2(agg2p, degp, hh, W2, Wm,
                        b2.reshape(1, d), bm.reshape(1, 1))
  score = _make_score(np_, ep)(s0p, s1p, srcp, dstp)
  return score[:e].reshape(e, 1)
